# baseline (speedup 1.0000x reference)
import jax
import jax.numpy as jnp
from jax.experimental import pallas as pl
from jax.experimental.pallas import tpu as pltpu

D_MODEL = 1024
D_STATE = 64
D_CONV = 4
HEADDIM = 128
NHEADS = 16
D_INNER = 2048
CONV_DIM = 2176
BATCH = 2
SEQLEN = 2048
Q = 256
SUB = 2
RMS_EPS = 1e-5
NEG_BIG = -1e30


def _softplus(x):
    return jnp.maximum(x, 0.0) + jnp.log1p(jnp.exp(-jnp.abs(x)))


def _mamba_kernel(u_ref, wz_ref, wdt_ref, wdt_o_ref, cw_ref, cb_ref,
                  dtb_row_ref, dtb_col_ref, a_row_ref, a_col_ref, d_row_ref,
                  nw_ref, wo_ref, o_ref, xbuf_ref, state_ref):
    t = pl.program_id(1)

    @pl.when(t == 0)
    def _():
        xbuf_ref[0:D_CONV - 1] = jnp.zeros((D_CONV - 1, CONV_DIM), jnp.float32)
        state_ref[...] = jnp.zeros_like(state_ref)

    ri = jax.lax.broadcasted_iota(jnp.int32, (Q, Q), 0)
    ci = jax.lax.broadcasted_iota(jnp.int32, (Q, Q), 1)
    causal = ci <= ri
    trif = jnp.where(causal, 1.0, 0.0)
    negmask = jnp.where(causal, 0.0, NEG_BIG)
    he = jax.lax.broadcasted_iota(jnp.int32, (16, D_INNER), 0)
    le = jax.lax.broadcasted_iota(jnp.int32, (16, D_INNER), 1)
    expand = jnp.where(he == (le >> 7), 1.0, 0.0).astype(jnp.bfloat16)
    hi16 = jax.lax.broadcasted_iota(jnp.int32, (16, Q), 0)

    for sub in range(SUB):
        _chunk(u_ref, wz_ref, wdt_ref, wdt_o_ref, cw_ref, cb_ref,
               dtb_row_ref, dtb_col_ref, a_row_ref, a_col_ref, d_row_ref,
               nw_ref, wo_ref, o_ref, xbuf_ref, state_ref,
               sub, trif, negmask, expand, hi16)


def _chunk(u_ref, wz_ref, wdt_ref, wdt_o_ref, cw_ref, cb_ref,
           dtb_row_ref, dtb_col_ref, a_row_ref, a_col_ref, d_row_ref,
           nw_ref, wo_ref, o_ref, xbuf_ref, state_ref,
           sub, trif, negmask, expand, hi16):
    uv = u_ref[0, sub * Q:(sub + 1) * Q, :]
    uvb = uv.astype(jnp.bfloat16)

    cdims = (((1,), (1,)), ((), ()))
    z = jax.lax.dot_general(uvb, wz_ref[0:D_INNER], cdims,
                            preferred_element_type=jnp.float32)
    xbc = jax.lax.dot_general(uvb, wz_ref[D_INNER:D_INNER + CONV_DIM], cdims,
                              preferred_element_type=jnp.float32)
    dtr = jnp.dot(uv, wdt_ref[...], preferred_element_type=jnp.float32)
    dtr_t = jax.lax.dot_general(wdt_o_ref[...], uv, (((1,), (1,)), ((), ())),
                                preferred_element_type=jnp.float32)

    xbuf_ref[D_CONV - 1:D_CONV - 1 + Q] = xbc
    conv = (cw_ref[0:1] * xbuf_ref[0:Q] + cw_ref[1:2] * xbuf_ref[1:1 + Q]
            + cw_ref[2:3] * xbuf_ref[2:2 + Q] + cw_ref[3:4] * xbuf_ref[3:3 + Q]
            + cb_ref[...])
    xbuf_ref[0:D_CONV - 1] = xbuf_ref[Q:Q + D_CONV - 1]
    xbc_a = conv * jax.nn.sigmoid(conv)

    x_full = xbc_a[:, :D_INNER]
    bmat = xbc_a[:, D_INNER:D_INNER + D_STATE]
    cmat = xbc_a[:, D_INNER + D_STATE:]

    dt = _softplus(dtr + dtb_row_ref[...])
    dt_t = _softplus(dtr_t + dtb_col_ref[...])
    a = dt * a_row_ref[...]
    a_t = dt_t * a_col_ref[...]

    s = jnp.dot(trif, a, preferred_element_type=jnp.float32)
    s_t = jax.lax.dot_general(a_t, trif, (((1,), (1,)), ((), ())),
                              preferred_element_type=jnp.float32)

    es16 = jnp.exp(s)
    es_last = es16[Q - 1:Q, :]
    wd16 = jnp.exp(s[Q - 1:Q, :] - s) * dt

    dt_exp = jnp.dot(dt.astype(jnp.bfloat16), expand,
                     preferred_element_type=jnp.float32).astype(jnp.bfloat16)
    wd_exp = jnp.dot(wd16.astype(jnp.bfloat16), expand,
                     preferred_element_type=jnp.float32).astype(jnp.bfloat16)
    es_exp = jnp.dot(es16.astype(jnp.bfloat16), expand,
                     preferred_element_type=jnp.float32)

    x_b = x_full.astype(jnp.bfloat16)
    dtx_b = x_b * dt_exp
    xw_b = x_b * wd_exp

    bmat_b = bmat.astype(jnp.bfloat16)
    cmat_b = cmat.astype(jnp.bfloat16)
    g = jax.lax.dot_general(cmat_b, bmat_b, (((1,), (1,)), ((), ())),
                            preferred_element_type=jnp.float32)

    ys = []
    for h in range(NHEADS):
        lo, hihd = h * HEADDIM, (h + 1) * HEADDIM
        w1h = jnp.where(hi16 == h, 1.0, 0.0)
        col_s = jnp.dot(s, w1h, preferred_element_type=jnp.float32)
        row_s = s_t[h:h + 1, :]
        m = jnp.exp(col_s - row_s + negmask)
        p = (g * m).astype(jnp.bfloat16)
        s_h = state_ref[h]
        y_h = (jnp.dot(p, dtx_b[:, lo:hihd], preferred_element_type=jnp.float32)
               + es_exp[:, lo:hihd]
               * jnp.dot(cmat_b, s_h.astype(jnp.bfloat16),
                         preferred_element_type=jnp.float32))
        contrib = jax.lax.dot_general(bmat_b, xw_b[:, lo:hihd],
                                      (((0,), (0,)), ((), ())),
                                      preferred_element_type=jnp.float32)
        state_ref[h] = s_h * es_last[0:1, h:h + 1] + contrib
        ys.append(y_h)

    y = jnp.concatenate(ys, axis=1) + x_full * d_row_ref[...]

    yg = y * (z * jax.nn.sigmoid(z))
    ms = jnp.mean(yg * yg, axis=-1, keepdims=True)
    yn = yg * jax.lax.rsqrt(ms + RMS_EPS) * nw_ref[...]

    o_ref[0, sub * Q:(sub + 1) * Q, :] = jax.lax.dot_general(
        yn.astype(jnp.bfloat16), wo_ref[...], cdims,
        preferred_element_type=jnp.float32)


def kernel(u, in_proj_w, conv_w, conv_b, dt_bias, A_log, D_param, norm_w, out_proj_w):
    f32 = jnp.float32
    bf16 = jnp.bfloat16
    wzx = in_proj_w[:D_INNER + CONV_DIM].astype(bf16)
    wdt_o = in_proj_w[D_INNER + CONV_DIM:]
    wdt_t = wdt_o.T
    cw4 = conv_w[:, 0, :].T
    cb = conv_b.reshape(1, CONV_DIM)
    dtb_row = dt_bias.reshape(1, NHEADS)
    dtb_col = dt_bias.reshape(NHEADS, 1)
    a_row = (-jnp.exp(A_log)).reshape(1, NHEADS)
    a_col = a_row.reshape(NHEADS, 1)
    d_row = jnp.repeat(D_param, HEADDIM).reshape(1, D_INNER)
    nw = norm_w.reshape(1, D_INNER)
    wo_b = out_proj_w.astype(bf16)

    nt = SEQLEN // (Q * SUB)
    grid = (BATCH, nt)
    full = lambda shp: pl.BlockSpec(shp, lambda b, t: (0,) * len(shp))
    out = pl.pallas_call(
        _mamba_kernel,
        grid=grid,
        in_specs=[
            pl.BlockSpec((1, Q * SUB, D_MODEL), lambda b, t: (b, t, 0)),
            full((D_INNER + CONV_DIM, D_MODEL)),
            full((D_MODEL, NHEADS)),
            full((NHEADS, D_MODEL)),
            full((D_CONV, CONV_DIM)),
            full((1, CONV_DIM)),
            full((1, NHEADS)),
            full((NHEADS, 1)),
            full((1, NHEADS)),
            full((NHEADS, 1)),
            full((1, D_INNER)),
            full((1, D_INNER)),
            full((D_MODEL, D_INNER)),
        ],
        out_specs=pl.BlockSpec((1, Q * SUB, D_MODEL), lambda b, t: (b, t, 0)),
        out_shape=jax.ShapeDtypeStruct((BATCH, SEQLEN, D_MODEL), f32),
        scratch_shapes=[
            pltpu.VMEM((Q + 8, CONV_DIM), f32),
            pltpu.VMEM((NHEADS, D_STATE, HEADDIM), f32),
        ],
        compiler_params=pltpu.CompilerParams(
            dimension_semantics=("parallel", "arbitrary"),
            vmem_limit_bytes=100 * 1024 * 1024,
        ),
    )(u.astype(f32), wzx, wdt_t, wdt_o, cw4, cb, dtb_row, dtb_col,
      a_row, a_col, d_row, nw, wo_b)
    return out


# speedup vs baseline: 1.0795x; 1.0795x over previous
import jax
import jax.numpy as jnp
from jax.experimental import pallas as pl
from jax.experimental.pallas import tpu as pltpu

D_MODEL = 1024
D_STATE = 64
D_CONV = 4
HEADDIM = 128
NHEADS = 16
D_INNER = 2048
CONV_DIM = 2176
D_IN_PROJ = 4240
BATCH = 2
SEQLEN = 2048
Q = 256
RMS_EPS = 1e-5
NEG_BIG = -1e30


def _softplus(x):
    return jnp.maximum(x, 0.0) + jnp.log1p(jnp.exp(-jnp.abs(x)))


def _mamba_kernel(u_ref, w_ref, cw_ref, cb_ref,
                  dtb_row_ref, dtb_col_ref, a_row_ref, a_col_ref, d_row_ref,
                  nw_ref, wo_ref, o_ref, xbuf_ref, state_ref, wb_ref, wob_ref):
    t = pl.program_id(1)

    @pl.when(t == 0)
    def _():
        xbuf_ref[0:D_CONV - 1] = jnp.zeros((D_CONV - 1, CONV_DIM), jnp.float32)
        state_ref[...] = jnp.zeros_like(state_ref)
        for r0 in range(0, D_INNER + CONV_DIM, 528):
            wb_ref[r0:r0 + 528] = w_ref[r0:r0 + 528].astype(jnp.bfloat16)
        for r0 in range(0, D_MODEL, 512):
            wob_ref[r0:r0 + 512] = wo_ref[r0:r0 + 512].astype(jnp.bfloat16)

    uv = u_ref[0]
    uvb = uv.astype(jnp.bfloat16)

    cdims = (((1,), (1,)), ((), ()))
    z = jax.lax.dot_general(uvb, wb_ref[0:D_INNER], cdims,
                            preferred_element_type=jnp.float32)
    xbc = jax.lax.dot_general(uvb, wb_ref[D_INNER:D_INNER + CONV_DIM], cdims,
                              preferred_element_type=jnp.float32)
    wdt = w_ref[D_INNER + CONV_DIM:D_IN_PROJ]
    dtr = jax.lax.dot_general(uv, wdt, cdims,
                              preferred_element_type=jnp.float32)
    dtr_t = jax.lax.dot_general(wdt, uv, (((1,), (1,)), ((), ())),
                                preferred_element_type=jnp.float32)

    xbuf_ref[D_CONV - 1:D_CONV - 1 + Q] = xbc
    conv = (cw_ref[0:1] * xbuf_ref[0:Q] + cw_ref[1:2] * xbuf_ref[1:1 + Q]
            + cw_ref[2:3] * xbuf_ref[2:2 + Q] + cw_ref[3:4] * xbuf_ref[3:3 + Q]
            + cb_ref[...])
    xbuf_ref[0:D_CONV - 1] = xbuf_ref[Q:Q + D_CONV - 1]
    xbc_a = conv * jax.nn.sigmoid(conv)

    x_full = xbc_a[:, :D_INNER]
    bmat = xbc_a[:, D_INNER:D_INNER + D_STATE]
    cmat = xbc_a[:, D_INNER + D_STATE:]

    dt = _softplus(dtr + dtb_row_ref[...])
    dt_t = _softplus(dtr_t + dtb_col_ref[...])
    a = dt * a_row_ref[...]
    a_t = dt_t * a_col_ref[...]

    ri = jax.lax.broadcasted_iota(jnp.int32, (Q, Q), 0)
    ci = jax.lax.broadcasted_iota(jnp.int32, (Q, Q), 1)
    causal = ci <= ri
    trif = jnp.where(causal, 1.0, 0.0)
    negmask = jnp.where(causal, 0.0, NEG_BIG)

    s = jnp.dot(trif, a, preferred_element_type=jnp.float32)
    s_t = jax.lax.dot_general(a_t, trif, (((1,), (1,)), ((), ())),
                              preferred_element_type=jnp.float32)

    es16 = jnp.exp(s)
    es_last = es16[Q - 1:Q, :]
    wd16 = jnp.exp(s[Q - 1:Q, :] - s) * dt

    he = jax.lax.broadcasted_iota(jnp.int32, (16, D_INNER), 0)
    le = jax.lax.broadcasted_iota(jnp.int32, (16, D_INNER), 1)
    expand = jnp.where(he == (le >> 7), 1.0, 0.0).astype(jnp.bfloat16)
    dt_exp = jnp.dot(dt.astype(jnp.bfloat16), expand,
                     preferred_element_type=jnp.float32).astype(jnp.bfloat16)
    wd_exp = jnp.dot(wd16.astype(jnp.bfloat16), expand,
                     preferred_element_type=jnp.float32).astype(jnp.bfloat16)
    es_exp = jnp.dot(es16.astype(jnp.bfloat16), expand,
                     preferred_element_type=jnp.float32)

    x_b = x_full.astype(jnp.bfloat16)
    dtx_b = x_b * dt_exp
    xw_b = x_b * wd_exp

    bmat_b = bmat.astype(jnp.bfloat16)
    cmat_b = cmat.astype(jnp.bfloat16)
    g = jax.lax.dot_general(cmat_b, bmat_b, (((1,), (1,)), ((), ())),
                            preferred_element_type=jnp.float32)

    hi16 = jax.lax.broadcasted_iota(jnp.int32, (16, Q), 0)
    ys = []
    for h in range(NHEADS):
        lo, hihd = h * HEADDIM, (h + 1) * HEADDIM
        w1h = jnp.where(hi16 == h, 1.0, 0.0)
        col_s = jnp.dot(s, w1h, preferred_element_type=jnp.float32)
        row_s = s_t[h:h + 1, :]
        m = jnp.exp(col_s - row_s + negmask)
        p = (g * m).astype(jnp.bfloat16)
        s_h = state_ref[h]
        y_h = (jnp.dot(p, dtx_b[:, lo:hihd], preferred_element_type=jnp.float32)
               + es_exp[:, lo:hihd]
               * jnp.dot(cmat_b, s_h.astype(jnp.bfloat16),
                         preferred_element_type=jnp.float32))
        contrib = jax.lax.dot_general(bmat_b, xw_b[:, lo:hihd],
                                      (((0,), (0,)), ((), ())),
                                      preferred_element_type=jnp.float32)
        state_ref[h] = s_h * es_last[0:1, h:h + 1] + contrib
        ys.append(y_h)

    y = jnp.concatenate(ys, axis=1) + x_full * d_row_ref[...]

    yg = y * (z * jax.nn.sigmoid(z))
    ms = jnp.mean(yg * yg, axis=-1, keepdims=True)
    yn = yg * jax.lax.rsqrt(ms + RMS_EPS) * nw_ref[...]

    o_ref[0] = jax.lax.dot_general(yn.astype(jnp.bfloat16), wob_ref[...],
                                   cdims, preferred_element_type=jnp.float32)


def kernel(u, in_proj_w, conv_w, conv_b, dt_bias, A_log, D_param, norm_w, out_proj_w):
    f32 = jnp.float32
    bf16 = jnp.bfloat16
    cw4 = conv_w[:, 0, :].T
    cb = conv_b.reshape(1, CONV_DIM)
    dtb_row = dt_bias.reshape(1, NHEADS)
    dtb_col = dt_bias.reshape(NHEADS, 1)
    a_row = (-jnp.exp(A_log)).reshape(1, NHEADS)
    a_col = a_row.reshape(NHEADS, 1)
    d_row = jnp.repeat(D_param, HEADDIM).reshape(1, D_INNER)
    nw = norm_w.reshape(1, D_INNER)

    nt = SEQLEN // Q
    grid = (BATCH, nt)
    full = lambda shp: pl.BlockSpec(shp, lambda b, t: (0,) * len(shp))
    out = pl.pallas_call(
        _mamba_kernel,
        grid=grid,
        in_specs=[
            pl.BlockSpec((1, Q, D_MODEL), lambda b, t: (b, t, 0)),
            full((D_IN_PROJ, D_MODEL)),
            full((D_CONV, CONV_DIM)),
            full((1, CONV_DIM)),
            full((1, NHEADS)),
            full((NHEADS, 1)),
            full((1, NHEADS)),
            full((NHEADS, 1)),
            full((1, D_INNER)),
            full((1, D_INNER)),
            full((D_MODEL, D_INNER)),
        ],
        out_specs=pl.BlockSpec((1, Q, D_MODEL), lambda b, t: (b, t, 0)),
        out_shape=jax.ShapeDtypeStruct((BATCH, SEQLEN, D_MODEL), f32),
        scratch_shapes=[
            pltpu.VMEM((Q + 8, CONV_DIM), f32),
            pltpu.VMEM((NHEADS, D_STATE, HEADDIM), f32),
            pltpu.VMEM((D_INNER + CONV_DIM, D_MODEL), bf16),
            pltpu.VMEM((D_MODEL, D_INNER), bf16),
        ],
        compiler_params=pltpu.CompilerParams(
            dimension_semantics=("parallel", "arbitrary"),
            vmem_limit_bytes=100 * 1024 * 1024,
        ),
    )(u.astype(f32), in_proj_w, cw4, cb, dtb_row, dtb_col,
      a_row, a_col, d_row, nw, out_proj_w)
    return out
